# baseline (speedup 1.0000x reference)
import functools

import jax
import jax.numpy as jnp
from jax import lax
from jax.experimental import pallas as pl
from jax.experimental.pallas import tpu as pltpu
from jax.experimental.pallas import tpu_sc as plsc

N = 10000
E = 320000
D = 128
NEG_SLOPE = 0.01

NC = 2
NS = 16
NW = NC * NS
EPW = E // NW
C = 80
NCH = EPW // C
GRP = 5
NG = NCH // GRP
RPS = 624
RTAIL = N - NS * RPS
LG = D // 16


def _sc_kernel():
    mesh = plsc.VectorSubcoreMesh(core_axis_name="c", subcore_axis_name="s")

    @functools.partial(
        pl.kernel,
        out_type=jax.ShapeDtypeStruct((NC, N, D), jnp.float32),
        mesh=mesh,
        scratch_types=[
            pltpu.VMEM((2 * GRP, C), jnp.int32),
            pltpu.VMEM((2 * GRP, C), jnp.int32),
            pltpu.VMEM((2 * GRP, C), jnp.float32),
            pltpu.VMEM((C,), jnp.float32),
            pltpu.VMEM((3, C, D), jnp.float32),
            pltpu.VMEM_SHARED((N, D), jnp.float32),
            pltpu.SemaphoreType.DMA((3,)),
            pltpu.SemaphoreType.DMA((3,)),
        ],
        compiler_params=pltpu.CompilerParams(needs_layout_passes=False),
    )
    def k(z_hbm, src_hbm, dst_hbm, en_hbm, out_hbm,
          src_v, dst_v, en_v, scale_v, rows_v, h_shared, gsem, ssem):
        cid = lax.axis_index("c")
        sid = lax.axis_index("s")
        wid = cid * NS + sid

        zrow = jnp.zeros((16,), jnp.float32)

        def zero_body(i, _):
            rows_v[0, i // LG, pl.ds((i % LG) * 16, 16)] = zrow
            return 0

        lax.fori_loop(0, C * LG, zero_body, 0)
        for t in range(RPS // C):
            pltpu.sync_copy(rows_v.at[0], h_shared.at[pl.ds(sid * RPS + t * C, C)])
        rem = RPS - (RPS // C) * C
        pltpu.sync_copy(rows_v.at[0, pl.ds(0, rem)],
                        h_shared.at[pl.ds(sid * RPS + (RPS // C) * C, rem)])

        @pl.when(sid == NS - 1)
        def _zero_tail():
            pltpu.sync_copy(rows_v.at[0, pl.ds(0, RTAIL)],
                            h_shared.at[pl.ds(NS * RPS, RTAIL)])

        plsc.subcore_barrier()

        def stage_group(t, half):
            sl = pl.ds(half * GRP, GRP)
            pltpu.sync_copy(src_hbm.at[wid, t], src_v.at[sl])
            pltpu.sync_copy(dst_hbm.at[wid, t], dst_v.at[sl])
            pltpu.sync_copy(en_hbm.at[wid, t], en_v.at[sl])

        def gather(row, b):
            pltpu.async_copy(z_hbm.at[src_v.at[row]], rows_v.at[b], gsem.at[b])

        def compute(kk, b):
            row = lax.rem(kk, 2 * GRP)

            def scale_body(i, _):
                sl = pl.ds(i * 16, 16)
                scale_v[sl] = en_v[row, sl]
                return 0

            lax.fori_loop(0, C // 16, scale_body, 0)

            def edge_body(e, _):
                bv = plsc.load_gather(scale_v, [jnp.full((16,), e, jnp.int32)])
                for j in range(LG):
                    rows_v[b, e, pl.ds(j * 16, 16)] = (
                        rows_v[b, e, pl.ds(j * 16, 16)] * bv)
                return 0

            lax.fori_loop(0, C, edge_body, 0)

        stage_group(0, 0)
        gather(0, 0)
        gather(1, 1)

        def loop_body(kk, _):
            b = lax.rem(kk, 3)

            pltpu.make_async_copy(z_hbm.at[pl.ds(0, C)],
                                  rows_v.at[b], gsem.at[b]).wait()
            compute(kk, b)
            pltpu.async_copy(rows_v.at[b],
                             h_shared.at[dst_v.at[lax.rem(kk, 2 * GRP)]],
                             ssem.at[b], add=True)

            @pl.when(jnp.logical_and(lax.rem(kk, GRP) == GRP - 2,
                                     kk + 2 < NCH))
            def _stage_next():
                t1 = kk // GRP + 1
                stage_group(t1, lax.rem(t1, 2))

            @pl.when(kk + 2 < NCH)
            def _gather_next():
                b2 = lax.rem(kk + 2, 3)

                @pl.when(kk >= 1)
                def _drain_prev():
                    pltpu.make_async_copy(rows_v.at[b2],
                                          h_shared.at[pl.ds(0, C)],
                                          ssem.at[b2]).wait()

                gather(lax.rem(kk + 2, 2 * GRP), b2)

            return 0

        lax.fori_loop(0, NCH, loop_body, 0)

        for d in range(NCH - 3, NCH):
            pltpu.make_async_copy(rows_v.at[d % 3],
                                  h_shared.at[pl.ds(0, C)],
                                  ssem.at[d % 3]).wait()

        plsc.subcore_barrier()

        pltpu.sync_copy(h_shared.at[pl.ds(sid * RPS, RPS)],
                        out_hbm.at[cid, pl.ds(sid * RPS, RPS)])

        @pl.when(sid == NS - 1)
        def _write_tail():
            pltpu.sync_copy(h_shared.at[pl.ds(NS * RPS, RTAIL)],
                            out_hbm.at[cid, pl.ds(NS * RPS, RTAIL)])

    return k


_sc_run = _sc_kernel()


def _tc_scale_body(x_ref, nn_ref, z_ref):
    z_ref[...] = x_ref[...] * nn_ref[...]


_tc_scale = pl.pallas_call(
    _tc_scale_body,
    out_shape=jax.ShapeDtypeStruct((N, D), jnp.float32),
)


def _tc_finish_body(x_ref, p_ref, nn_ref, w_ref, b_ref, o_ref):
    xmax = jnp.max(x_ref[...], axis=0, keepdims=True)
    s = (p_ref[0] + p_ref[1]) * nn_ref[...]
    hmax = jnp.max(s, axis=0, keepdims=True)
    r = 0.5 * (xmax + hmax)
    out = lax.dot_general(r, w_ref[...], (((1,), (1,)), ((), ())),
                          preferred_element_type=jnp.float32) + b_ref[...]
    o_ref[...] = jnp.where(out > 0, out, NEG_SLOPE * out)


_tc_finish = pl.pallas_call(
    _tc_finish_body,
    out_shape=jax.ShapeDtypeStruct((1, D), jnp.float32),
)


def kernel(x, node_norm, edge_norm, W, b, edge_index):
    nn2 = node_norm.reshape(N, 1)
    z = _tc_scale(x, nn2)
    src = edge_index[0].reshape(NW, NG, GRP, C)
    dst = edge_index[1].reshape(NW, NG, GRP, C)
    en = edge_norm.reshape(NW, NG, GRP, C)
    partials = _sc_run(z, src, dst, en)
    return _tc_finish(x, partials, nn2, W, b.reshape(1, D))


# speedup vs baseline: 2.1628x; 2.1628x over previous
import functools

import jax
import jax.numpy as jnp
from jax import lax
from jax.experimental import pallas as pl
from jax.experimental.pallas import tpu as pltpu
from jax.experimental.pallas import tpu_sc as plsc

N = 10000
E = 320000
D = 128
NEG_SLOPE = 0.01

NC = 2
NS = 16
NW = NC * NS
EPW = E // NW
C = 80
NCH = EPW // C
GRP = 5
NG = NCH // GRP
RPS = 624
RTAIL = N - NS * RPS
LG = D // 16


def _sc_kernel():
    mesh = plsc.VectorSubcoreMesh(core_axis_name="c", subcore_axis_name="s")

    @functools.partial(
        pl.kernel,
        out_type=jax.ShapeDtypeStruct((NC, N, D), jnp.float32),
        mesh=mesh,
        scratch_types=[
            pltpu.VMEM((2 * GRP, C), jnp.int32),
            pltpu.VMEM((2 * GRP, C), jnp.int32),
            pltpu.VMEM((2 * GRP, C), jnp.float32),
            pltpu.VMEM((C,), jnp.float32),
            pltpu.VMEM((2, C, D), jnp.float32),
            pltpu.VMEM_SHARED((N, D), jnp.float32),
            pltpu.SemaphoreType.DMA((2,)),
        ],
        compiler_params=pltpu.CompilerParams(needs_layout_passes=False),
    )
    def k(z_hbm, src_hbm, dst_hbm, en_hbm, out_hbm,
          src_v, dst_v, en_v, scale_v, rows_v, h_shared, gsem):
        cid = lax.axis_index("c")
        sid = lax.axis_index("s")
        wid = cid * NS + sid

        zrow = jnp.zeros((16,), jnp.float32)

        def zero_body(i, _):
            rows_v[0, i // LG, pl.ds((i % LG) * 16, 16)] = zrow
            return 0

        lax.fori_loop(0, C * LG, zero_body, 0)
        for t in range(RPS // C):
            pltpu.sync_copy(rows_v.at[0], h_shared.at[pl.ds(sid * RPS + t * C, C)])
        rem = RPS - (RPS // C) * C
        pltpu.sync_copy(rows_v.at[0, pl.ds(0, rem)],
                        h_shared.at[pl.ds(sid * RPS + (RPS // C) * C, rem)])

        @pl.when(sid == NS - 1)
        def _zero_tail():
            pltpu.sync_copy(rows_v.at[0, pl.ds(0, RTAIL)],
                            h_shared.at[pl.ds(NS * RPS, RTAIL)])

        plsc.subcore_barrier()

        def stage_group(t, half):
            sl = pl.ds(half * GRP, GRP)
            pltpu.sync_copy(src_hbm.at[wid, t], src_v.at[sl])
            pltpu.sync_copy(dst_hbm.at[wid, t], dst_v.at[sl])
            pltpu.sync_copy(en_hbm.at[wid, t], en_v.at[sl])

        def gather(row, b):
            pltpu.async_copy(z_hbm.at[src_v.at[row]], rows_v.at[b], gsem.at[b])

        stage_group(0, 0)
        gather(0, 0)

        def group_body(t, _):
            tb = lax.rem(t, 2)
            ntb = 1 - tb

            @pl.when(t + 1 < NG)
            def _stage_next():
                stage_group(t + 1, ntb)

            for g in range(GRP):
                b = lax.rem(t * GRP + g, 2)
                nb = 1 - b

                pltpu.make_async_copy(z_hbm.at[pl.ds(0, C)],
                                      rows_v.at[b], gsem.at[b]).wait()

                if g < GRP - 1:
                    gather(tb * GRP + g + 1, nb)
                else:
                    @pl.when(t + 1 < NG)
                    def _gather_next_group():
                        gather(ntb * GRP, nb)

                def scale_body(i, _):
                    sl = pl.ds(i * 16, 16)
                    scale_v[sl] = en_v[tb * GRP + g, sl]
                    return 0

                lax.fori_loop(0, C // 16, scale_body, 0)

                @plsc.parallel_loop(0, C, step=1, unroll=4)
                def edge_body(e):
                    bv = plsc.load_gather(scale_v, [jnp.full((16,), e, jnp.int32)])
                    for j in range(LG):
                        rows_v[b, e, pl.ds(j * 16, 16)] = (
                            rows_v[b, e, pl.ds(j * 16, 16)] * bv)

                pltpu.sync_copy(rows_v.at[b],
                                h_shared.at[dst_v.at[tb * GRP + g]], add=True)
            return 0

        lax.fori_loop(0, NG, group_body, 0)

        plsc.subcore_barrier()

        pltpu.sync_copy(h_shared.at[pl.ds(sid * RPS, RPS)],
                        out_hbm.at[cid, pl.ds(sid * RPS, RPS)])

        @pl.when(sid == NS - 1)
        def _write_tail():
            pltpu.sync_copy(h_shared.at[pl.ds(NS * RPS, RTAIL)],
                            out_hbm.at[cid, pl.ds(NS * RPS, RTAIL)])

    return k


_sc_run = _sc_kernel()


def _tc_scale_body(x_ref, nn_ref, z_ref):
    z_ref[...] = x_ref[...] * nn_ref[...]


_tc_scale = pl.pallas_call(
    _tc_scale_body,
    out_shape=jax.ShapeDtypeStruct((N, D), jnp.float32),
)


def _tc_finish_body(x_ref, p_ref, nn_ref, w_ref, b_ref, o_ref):
    xmax = jnp.max(x_ref[...], axis=0, keepdims=True)
    s = (p_ref[0] + p_ref[1]) * nn_ref[...]
    hmax = jnp.max(s, axis=0, keepdims=True)
    r = 0.5 * (xmax + hmax)
    out = lax.dot_general(r, w_ref[...], (((1,), (1,)), ((), ())),
                          preferred_element_type=jnp.float32) + b_ref[...]
    o_ref[...] = jnp.where(out > 0, out, NEG_SLOPE * out)


_tc_finish = pl.pallas_call(
    _tc_finish_body,
    out_shape=jax.ShapeDtypeStruct((1, D), jnp.float32),
)


def kernel(x, node_norm, edge_norm, W, b, edge_index):
    nn2 = node_norm.reshape(N, 1)
    z = _tc_scale(x, nn2)
    src = edge_index[0].reshape(NW, NG, GRP, C)
    dst = edge_index[1].reshape(NW, NG, GRP, C)
    en = edge_norm.reshape(NW, NG, GRP, C)
    partials = _sc_run(z, src, dst, en)
    return _tc_finish(x, partials, nn2, W, b.reshape(1, D))
